# baseline (speedup 1.0000x reference)
import jax
import jax.numpy as jnp
from jax.experimental import pallas as pl
from jax.experimental.pallas import tpu as pltpu


def _body(v_ref, x_ref, y_ref, o_ref, pos_ref):
    W = x_ref.shape[0]
    H = y_ref.shape[0]
    D = x_ref.shape[1]

    @pl.when(pl.program_id(0) == 0)
    def _build_pos():
        pos = x_ref[...][:, None, :] + y_ref[...][None, :, :]
        pos_ref[...] = pos.reshape(W * H, D)

    v = v_ref[0]
    o_ref[0] = v.T + pos_ref[...]


def kernel(visn_feats, x_table, y_table):
    B, C, W, H = visn_feats.shape
    S = W * H
    D = x_table.shape[1]
    v3 = visn_feats.reshape(B, C, S)
    return pl.pallas_call(
        _body,
        grid=(B,),
        in_specs=[
            pl.BlockSpec((1, C, S), lambda b: (b, 0, 0)),
            pl.BlockSpec((W, D), lambda b: (0, 0)),
            pl.BlockSpec((H, D), lambda b: (0, 0)),
        ],
        out_specs=pl.BlockSpec((1, S, C), lambda b: (b, 0, 0)),
        out_shape=jax.ShapeDtypeStruct((B, S, C), visn_feats.dtype),
        scratch_shapes=[pltpu.VMEM((S, D), visn_feats.dtype)],
    )(v3, x_table, y_table)


# speedup vs baseline: 1.0706x; 1.0706x over previous
import jax
import jax.numpy as jnp
from jax.experimental import pallas as pl
from jax.experimental.pallas import tpu as pltpu

_K = 4


def _body(v_ref, x_ref, y_ref, o_ref, vbuf, obuf, pos_ref, in_sems, out_sems):
    B = v_ref.shape[0]
    W = x_ref.shape[0]
    H = y_ref.shape[0]
    D = x_ref.shape[1]
    K = _K
    NB = B // K

    pos = x_ref[...][:, None, :] + y_ref[...][None, :, :]
    pos_ref[...] = pos.reshape(W * H, D)

    def in_cp(b, j):
        return pltpu.make_async_copy(v_ref.at[b], vbuf.at[j], in_sems.at[j])

    def out_cp(b, j):
        return pltpu.make_async_copy(obuf.at[j], o_ref.at[b], out_sems.at[j])

    for j in range(K):
        in_cp(j, j).start()

    def loop_body(i, carry):
        base = i * K
        for j in range(K):
            b = base + j
            in_cp(b, j).wait()

            @pl.when(i > 0)
            def _wait_prev_out():
                out_cp(b - K, j).wait()

            obuf[j] = vbuf[j].T + pos_ref[...]
            out_cp(b, j).start()

            @pl.when(i < NB - 1)
            def _prefetch_next():
                in_cp(b + K, j).start()

        return carry

    jax.lax.fori_loop(0, NB, loop_body, 0)

    for j in range(K):
        out_cp(B - K + j, j).wait()


def kernel(visn_feats, x_table, y_table):
    B, C, W, H = visn_feats.shape
    S = W * H
    D = x_table.shape[1]
    v3 = visn_feats.reshape(B, C, S)
    return pl.pallas_call(
        _body,
        in_specs=[
            pl.BlockSpec(memory_space=pltpu.MemorySpace.HBM),
            pl.BlockSpec(memory_space=pltpu.MemorySpace.VMEM),
            pl.BlockSpec(memory_space=pltpu.MemorySpace.VMEM),
        ],
        out_specs=pl.BlockSpec(memory_space=pltpu.MemorySpace.HBM),
        out_shape=jax.ShapeDtypeStruct((B, S, C), visn_feats.dtype),
        scratch_shapes=[
            pltpu.VMEM((_K, C, S), visn_feats.dtype),
            pltpu.VMEM((_K, S, C), visn_feats.dtype),
            pltpu.VMEM((S, D), visn_feats.dtype),
            pltpu.SemaphoreType.DMA((_K,)),
            pltpu.SemaphoreType.DMA((_K,)),
        ],
    )(v3, x_table[:W], y_table[:H])


# speedup vs baseline: 2.4048x; 2.2463x over previous
import jax
import jax.numpy as jnp
from jax.experimental import pallas as pl
from jax.experimental.pallas import tpu as pltpu


def _body(v_ref, x_ref, y_ref, o_ref, pos_ref):
    W = x_ref.shape[0]
    H = y_ref.shape[0]
    D = x_ref.shape[1]

    @pl.when(pl.program_id(0) == 0)
    def _build_pos():
        pos = x_ref[...][:, None, :] + y_ref[...][None, :, :]
        pos_ref[...] = pos.reshape(W * H, D)

    o_ref[0] = v_ref[0] + pos_ref[...]


def kernel(visn_feats, x_table, y_table):
    B, C, W, H = visn_feats.shape
    S = W * H
    D = x_table.shape[1]
    v = jnp.transpose(visn_feats, (0, 2, 3, 1)).reshape(B, S, C)
    return pl.pallas_call(
        _body,
        grid=(B,),
        in_specs=[
            pl.BlockSpec((1, S, C), lambda b: (b, 0, 0)),
            pl.BlockSpec((W, D), lambda b: (0, 0)),
            pl.BlockSpec((H, D), lambda b: (0, 0)),
        ],
        out_specs=pl.BlockSpec((1, S, C), lambda b: (b, 0, 0)),
        out_shape=jax.ShapeDtypeStruct((B, S, C), visn_feats.dtype),
        scratch_shapes=[pltpu.VMEM((S, D), visn_feats.dtype)],
    )(v, x_table, y_table)


# speedup vs baseline: 2.7667x; 1.1505x over previous
import jax
import jax.numpy as jnp
from jax.experimental import pallas as pl
from jax.experimental.pallas import tpu as pltpu


def _body(v_ref, x_ref, y_ref, o_ref, pos_ref):
    W = x_ref.shape[0]
    H = y_ref.shape[0]
    D = x_ref.shape[1]

    @pl.when(pl.program_id(0) == 0)
    def _build_pos():
        pos = x_ref[...][:, None, :] + y_ref[...][None, :, :]
        pos_ref[...] = pos.reshape(W * H, D)

    o_ref[...] = v_ref[...] + pos_ref[...][None]


def kernel(visn_feats, x_table, y_table):
    B, C, W, H = visn_feats.shape
    S = W * H
    D = x_table.shape[1]
    v = jnp.transpose(visn_feats, (0, 2, 3, 1)).reshape(B, S, C)
    return pl.pallas_call(
        _body,
        grid=(B // 2,),
        in_specs=[
            pl.BlockSpec((2, S, C), lambda b: (b, 0, 0)),
            pl.BlockSpec((W, D), lambda b: (0, 0)),
            pl.BlockSpec((H, D), lambda b: (0, 0)),
        ],
        out_specs=pl.BlockSpec((2, S, C), lambda b: (b, 0, 0)),
        out_shape=jax.ShapeDtypeStruct((B, S, C), visn_feats.dtype),
        scratch_shapes=[pltpu.VMEM((S, D), visn_feats.dtype)],
    )(v, x_table, y_table)


# speedup vs baseline: 2.8801x; 1.0410x over previous
import jax
import jax.numpy as jnp
from jax.experimental import pallas as pl
from jax.experimental.pallas import tpu as pltpu


def _body(v_ref, x_ref, y_ref, o_ref, pos_ref):
    W = x_ref.shape[0]
    H = y_ref.shape[0]
    D = x_ref.shape[1]

    @pl.when(pl.program_id(0) == 0)
    def _build_pos():
        pos = x_ref[...][:, None, :] + y_ref[...][None, :, :]
        pos_ref[...] = pos.reshape(W * H, D)

    o_ref[...] = v_ref[...] + pos_ref[...][None]


def kernel(visn_feats, x_table, y_table):
    B, C, W, H = visn_feats.shape
    S = W * H
    D = x_table.shape[1]
    v = jnp.transpose(visn_feats, (0, 2, 3, 1)).reshape(B, S, C)
    return pl.pallas_call(
        _body,
        grid=(B // 4,),
        in_specs=[
            pl.BlockSpec((4, S, C), lambda b: (b, 0, 0)),
            pl.BlockSpec((W, D), lambda b: (0, 0)),
            pl.BlockSpec((H, D), lambda b: (0, 0)),
        ],
        out_specs=pl.BlockSpec((4, S, C), lambda b: (b, 0, 0)),
        out_shape=jax.ShapeDtypeStruct((B, S, C), visn_feats.dtype),
        scratch_shapes=[pltpu.VMEM((S, D), visn_feats.dtype)],
    )(v, x_table, y_table)


# speedup vs baseline: 3.0175x; 1.0477x over previous
import jax
import jax.numpy as jnp
from jax.experimental import pallas as pl
from jax.experimental.pallas import tpu as pltpu


def _body(v_ref, x_ref, y_ref, o_ref, pos_ref):
    W = x_ref.shape[0]
    H = y_ref.shape[0]
    D = x_ref.shape[1]

    @pl.when(pl.program_id(0) == 0)
    def _build_pos():
        pos = x_ref[...][:, None, :] + y_ref[...][None, :, :]
        pos_ref[...] = pos.reshape(W * H, D)

    o_ref[...] = v_ref[...] + pos_ref[...][None]


def kernel(visn_feats, x_table, y_table):
    B, C, W, H = visn_feats.shape
    S = W * H
    D = x_table.shape[1]
    v = jnp.transpose(visn_feats, (0, 2, 3, 1)).reshape(B, S, C)
    return pl.pallas_call(
        _body,
        grid=(B // 8,),
        in_specs=[
            pl.BlockSpec((8, S, C), lambda b: (b, 0, 0)),
            pl.BlockSpec((W, D), lambda b: (0, 0)),
            pl.BlockSpec((H, D), lambda b: (0, 0)),
        ],
        out_specs=pl.BlockSpec((8, S, C), lambda b: (b, 0, 0)),
        out_shape=jax.ShapeDtypeStruct((B, S, C), visn_feats.dtype),
        scratch_shapes=[pltpu.VMEM((S, D), visn_feats.dtype)],
    )(v, x_table, y_table)
